# Initial kernel scaffold; baseline (speedup 1.0000x reference)
import jax
import jax.numpy as jnp
from jax.experimental import pallas as pl


def kernel(x, der_desc_wrt_pos):
    raise NotImplementedError("write your pallas kernel here")


# speedup vs baseline: 18.4392x; 18.4392x over previous
import functools

import jax
import jax.numpy as jnp
from jax import lax
from jax.experimental import pallas as pl
from jax.experimental.pallas import tpu as pltpu
from jax.experimental.pallas import tpu_sc as plsc

B, A, D = 16, 512, 1024
ROWS = B * A
ROWLEN = 3 * D
NW = 32
RPW = ROWS // NW
CH = 16
NCH = RPW // CH
OUTW = RPW * 3
GROUPS = D // 16


def _sc_body(der_hbm, x_hbm, out_hbm, buf, xv, outv):
    c = lax.axis_index("c")
    s = lax.axis_index("s")
    wid = s * 2 + c
    row0 = wid * RPW
    b = row0 // A

    pltpu.sync_copy(x_hbm.at[b], xv)

    iota = lax.iota(jnp.int32, 16)
    iota3 = iota * 3

    def chunk_body(ci, _):
        pltpu.sync_copy(
            der_hbm.at[pl.ds((row0 + ci * CH) * ROWLEN, CH * ROWLEN)], buf
        )

        def row_body(r, __):
            rbase = r * ROWLEN

            def g_body(g, accs):
                a0, a1, a2 = accs
                xvg = xv[pl.ds(g * 16, 16)]
                cidx = iota3 + (rbase + g * 48)
                v0 = plsc.load_gather(buf, [cidx])
                v1 = plsc.load_gather(buf, [cidx + 1])
                v2 = plsc.load_gather(buf, [cidx + 2])
                return (a0 + v0 * xvg, a1 + v1 * xvg, a2 + v2 * xvg)

            zero = jnp.zeros((16,), jnp.float32)
            a0, a1, a2 = lax.fori_loop(0, GROUPS, g_body, (zero, zero, zero))
            s0 = jnp.sum(a0)
            s1 = jnp.sum(a1)
            s2 = jnp.sum(a2)
            o3 = (ci * CH + r) * 3
            vec = jnp.where(iota == 0, s0 * s0,
                            jnp.where(iota == 1, s1 * s1, s2 * s2))
            plsc.store_scatter(outv, [o3 + iota % 3], vec, mask=iota < 3)
            return __

        lax.fori_loop(0, CH, row_body, 0)
        return _

    lax.fori_loop(0, NCH, chunk_body, 0)

    pltpu.sync_copy(outv, out_hbm.at[pl.ds(wid * OUTW, OUTW)])


@functools.partial(
    pl.kernel,
    out_type=jax.ShapeDtypeStruct((ROWS * 3,), jnp.float32),
    mesh=plsc.VectorSubcoreMesh(
        core_axis_name="c", subcore_axis_name="s", num_cores=2, num_subcores=16
    ),
    scratch_types=[
        pltpu.VMEM((CH * ROWLEN,), jnp.float32),
        pltpu.VMEM((D,), jnp.float32),
        pltpu.VMEM((OUTW,), jnp.float32),
    ],
    compiler_params=pltpu.CompilerParams(
        use_tc_tiling_on_sc=False, needs_layout_passes=False
    ),
)
def _sc_kernel(der_hbm, x_hbm, out_hbm, buf, xv, outv):
    _sc_body(der_hbm, x_hbm, out_hbm, buf, xv, outv)


def kernel(x, der_desc_wrt_pos):
    der2 = der_desc_wrt_pos.reshape(ROWS * ROWLEN)
    out_flat = _sc_kernel(der2, x)
    return out_flat.reshape(B, A * 3)


# speedup vs baseline: 5325.5057x; 288.8146x over previous
import functools

import jax
import jax.numpy as jnp
from jax import lax
from jax.experimental import pallas as pl
from jax.experimental.pallas import tpu as pltpu
from jax.experimental.pallas import tpu_sc as plsc

B, A, D = 16, 512, 1024
NW = 32
NBLK = B * 3 * (A // 8)
BLK = 8 * D
BPW = NBLK // NW
CHB = 6
NCH = BPW // CHB
OUTW = BPW * 8
GROUPS = D // 16


def _sc_body(der_hbm, x_hbm, out_hbm, buf0, buf1, xv, outv, sem0, sem1):
    c = lax.axis_index("c")
    s = lax.axis_index("s")
    wid = s * 2 + c
    base = wid * BPW * BLK
    b = wid // 2

    pltpu.sync_copy(x_hbm.at[b], xv)

    iota = lax.iota(jnp.int32, 16)
    bufs = (buf0, buf1)
    sems = (sem0, sem1)

    def src(ci):
        return der_hbm.at[pl.ds(base + ci * (CHB * BLK), CHB * BLK)]

    pltpu.make_async_copy(src(0), buf0, sem0).start()
    pltpu.make_async_copy(src(1), buf1, sem1).start()

    zero = jnp.zeros((16,), jnp.float32)

    def compute_chunk(ci, buf):
        for blk in range(CHB):
            boff = blk * BLK

            def g_body(t, accs):
                xvg = xv[pl.ds(t * 16, 16)]
                off = boff + (t >> 3) * 1024 + ((t & 7) << 4)
                return tuple(
                    accs[ai] + buf[pl.ds(off + ai * 128, 16)] * xvg
                    for ai in range(8)
                )

            accs = lax.fori_loop(0, GROUPS, g_body, (zero,) * 8)
            vec = zero
            for ai in range(8):
                sm = jnp.sum(accs[ai])
                vec = jnp.where(iota == ai, sm * sm, vec)
            obase = (ci * CHB + blk) * 8
            plsc.store_scatter(outv, [obase + (iota & 7)], vec, mask=iota < 8)

    def pair_body(cj, carry):
        for p in range(2):
            ci = cj * 2 + p
            pltpu.make_async_copy(src(ci), bufs[p], sems[p]).wait()
            compute_chunk(ci, bufs[p])
            nxt = ci + 2

            @pl.when(nxt < NCH)
            def _():
                pltpu.make_async_copy(src(nxt), bufs[p], sems[p]).start()

        return carry

    lax.fori_loop(0, NCH // 2, pair_body, 0)

    pltpu.sync_copy(outv, out_hbm.at[pl.ds(wid * OUTW, OUTW)])


@functools.partial(
    pl.kernel,
    out_type=jax.ShapeDtypeStruct((B * 3 * A,), jnp.float32),
    mesh=plsc.VectorSubcoreMesh(
        core_axis_name="c", subcore_axis_name="s", num_cores=2, num_subcores=16
    ),
    scratch_types=[
        pltpu.VMEM((CHB * BLK,), jnp.float32),
        pltpu.VMEM((CHB * BLK,), jnp.float32),
        pltpu.VMEM((D,), jnp.float32),
        pltpu.VMEM((OUTW,), jnp.float32),
        pltpu.SemaphoreType.DMA,
        pltpu.SemaphoreType.DMA,
    ],
    compiler_params=pltpu.CompilerParams(
        use_tc_tiling_on_sc=False, needs_layout_passes=False
    ),
)
def _sc_kernel(der_hbm, x_hbm, out_hbm, buf0, buf1, xv, outv, sem0, sem1):
    _sc_body(der_hbm, x_hbm, out_hbm, buf0, buf1, xv, outv, sem0, sem1)


def kernel(x, der_desc_wrt_pos):
    der_flat = (
        der_desc_wrt_pos.transpose(0, 3, 1, 2)
        .reshape(B * 3 * A // 8, 8, D // 128, 128)
        .transpose(0, 2, 1, 3)
        .reshape(-1)
    )
    out_phys = _sc_kernel(der_flat, x)
    return out_phys.reshape(B, 3, A).transpose(0, 2, 1).reshape(B, A * 3)


# speedup vs baseline: 5433.7968x; 1.0203x over previous
import functools

import jax
import jax.numpy as jnp
from jax import lax
from jax.experimental import pallas as pl
from jax.experimental.pallas import tpu as pltpu
from jax.experimental.pallas import tpu_sc as plsc

B, A, D = 16, 512, 1024
NW = 32
NBLK = B * 3 * (A // 8)
BLK = 8 * D
BPW = NBLK // NW
CHB = 6
NCH = BPW // CHB
OUTW = BPW * 8
GROUPS = D // 16


def _sc_body(der_hbm, x_hbm, out_hbm, buf0, buf1, xv, outv, sem0, sem1):
    c = lax.axis_index("c")
    s = lax.axis_index("s")
    wid = s * 2 + c
    base = wid * BPW * BLK
    b = wid // 2

    pltpu.sync_copy(x_hbm.at[b], xv)

    iota = lax.iota(jnp.int32, 16)
    bufs = (buf0, buf1)
    sems = (sem0, sem1)

    def src(ci):
        return der_hbm.at[pl.ds(base + ci * (CHB * BLK), CHB * BLK)]

    pltpu.make_async_copy(src(0), buf0, sem0).start()
    pltpu.make_async_copy(src(1), buf1, sem1).start()

    zero = jnp.zeros((16,), jnp.float32)

    def compute_chunk(ci, buf):
        for pair in range(CHB // 2):
            boff = pair * (2 * BLK)

            def g_body(t, accs):
                xvg = xv[pl.ds(t * 16, 16)]
                off = boff + (t >> 3) * 1024 + ((t & 7) << 4)
                return tuple(
                    accs[r] + buf[pl.ds(off + (r >> 3) * BLK + (r & 7) * 128, 16)] * xvg
                    for r in range(16)
                )

            accs = lax.fori_loop(0, GROUPS, g_body, (zero,) * 16)
            vec = zero
            for r in range(16):
                sm = jnp.sum(accs[r])
                vec = jnp.where(iota == r, sm * sm, vec)
            obase = (ci * CHB + pair * 2) * 8
            plsc.store_scatter(outv, [obase + iota], vec)

    def pair_body(cj, carry):
        for p in range(2):
            ci = cj * 2 + p
            pltpu.make_async_copy(src(ci), bufs[p], sems[p]).wait()
            compute_chunk(ci, bufs[p])
            nxt = ci + 2

            @pl.when(nxt < NCH)
            def _():
                pltpu.make_async_copy(src(nxt), bufs[p], sems[p]).start()

        return carry

    lax.fori_loop(0, NCH // 2, pair_body, 0)

    pltpu.sync_copy(outv, out_hbm.at[pl.ds(wid * OUTW, OUTW)])


@functools.partial(
    pl.kernel,
    out_type=jax.ShapeDtypeStruct((B * 3 * A,), jnp.float32),
    mesh=plsc.VectorSubcoreMesh(
        core_axis_name="c", subcore_axis_name="s", num_cores=2, num_subcores=16
    ),
    scratch_types=[
        pltpu.VMEM((CHB * BLK,), jnp.float32),
        pltpu.VMEM((CHB * BLK,), jnp.float32),
        pltpu.VMEM((D,), jnp.float32),
        pltpu.VMEM((OUTW,), jnp.float32),
        pltpu.SemaphoreType.DMA,
        pltpu.SemaphoreType.DMA,
    ],
    compiler_params=pltpu.CompilerParams(
        use_tc_tiling_on_sc=False, needs_layout_passes=False
    ),
)
def _sc_kernel(der_hbm, x_hbm, out_hbm, buf0, buf1, xv, outv, sem0, sem1):
    _sc_body(der_hbm, x_hbm, out_hbm, buf0, buf1, xv, outv, sem0, sem1)


def kernel(x, der_desc_wrt_pos):
    der_flat = (
        der_desc_wrt_pos.transpose(0, 3, 1, 2)
        .reshape(B * 3 * A // 8, 8, D // 128, 128)
        .transpose(0, 2, 1, 3)
        .reshape(-1)
    )
    out_phys = _sc_kernel(der_flat, x)
    return out_phys.reshape(B, 3, A).transpose(0, 2, 1).reshape(B, A * 3)


# speedup vs baseline: 6310.8123x; 1.1614x over previous
import functools

import jax
import jax.numpy as jnp
from jax import lax
from jax.experimental import pallas as pl
from jax.experimental.pallas import tpu as pltpu
from jax.experimental.pallas import tpu_sc as plsc

B, A, D = 16, 512, 1024
B_SC = 8
B_TC = B - B_SC
NW = 32
BLK = 8 * D
BPB = 3 * (A // 8)
BPW = B_SC * BPB // NW
CHB = 6
NCH = BPW // CHB
OUTW = BPW * 8
GROUPS = D // 16


def _sc_body(der_hbm, x_hbm, out_hbm, buf0, buf1, xv, outv, sem0, sem1):
    c = lax.axis_index("c")
    s = lax.axis_index("s")
    wid = s * 2 + c
    base = wid * BPW * BLK
    b = wid // (NW // B_SC)

    pltpu.sync_copy(x_hbm.at[b], xv)

    iota = lax.iota(jnp.int32, 16)
    bufs = (buf0, buf1)
    sems = (sem0, sem1)

    def src(ci):
        return der_hbm.at[pl.ds(base + ci * (CHB * BLK), CHB * BLK)]

    pltpu.make_async_copy(src(0), buf0, sem0).start()
    pltpu.make_async_copy(src(1), buf1, sem1).start()

    zero = jnp.zeros((16,), jnp.float32)

    def compute_chunk(ci, buf):
        for pair in range(CHB // 2):
            boff = pair * (2 * BLK)

            def g_body(t, accs):
                xvg = xv[pl.ds(t * 16, 16)]
                off = boff + (t >> 3) * 1024 + ((t & 7) << 4)
                return tuple(
                    accs[r] + buf[pl.ds(off + (r >> 3) * BLK + (r & 7) * 128, 16)] * xvg
                    for r in range(16)
                )

            accs = lax.fori_loop(0, GROUPS, g_body, (zero,) * 16)
            vec = zero
            for r in range(16):
                sm = jnp.sum(accs[r])
                vec = jnp.where(iota == r, sm * sm, vec)
            obase = (ci * CHB + pair * 2) * 8
            plsc.store_scatter(outv, [obase + iota], vec)

    def pair_body(cj, carry):
        for p in range(2):
            ci = cj * 2 + p
            pltpu.make_async_copy(src(ci), bufs[p], sems[p]).wait()
            compute_chunk(ci, bufs[p])
            nxt = ci + 2

            @pl.when(nxt < NCH)
            def _():
                pltpu.make_async_copy(src(nxt), bufs[p], sems[p]).start()

        return carry

    lax.fori_loop(0, NCH // 2, pair_body, 0)

    pltpu.sync_copy(outv, out_hbm.at[pl.ds(wid * OUTW, OUTW)])


@functools.partial(
    pl.kernel,
    out_type=jax.ShapeDtypeStruct((B_SC * 3 * A,), jnp.float32),
    mesh=plsc.VectorSubcoreMesh(
        core_axis_name="c", subcore_axis_name="s", num_cores=2, num_subcores=16
    ),
    scratch_types=[
        pltpu.VMEM((CHB * BLK,), jnp.float32),
        pltpu.VMEM((CHB * BLK,), jnp.float32),
        pltpu.VMEM((D,), jnp.float32),
        pltpu.VMEM((OUTW,), jnp.float32),
        pltpu.SemaphoreType.DMA,
        pltpu.SemaphoreType.DMA,
    ],
    compiler_params=pltpu.CompilerParams(
        use_tc_tiling_on_sc=False, needs_layout_passes=False
    ),
)
def _sc_kernel(der_hbm, x_hbm, out_hbm, buf0, buf1, xv, outv, sem0, sem1):
    _sc_body(der_hbm, x_hbm, out_hbm, buf0, buf1, xv, outv, sem0, sem1)


def _tc_body(der_ref, x_ref, out_ref):
    blk = der_ref[0, 0]
    xb = x_ref[0]
    sm = jnp.sum(blk * xb, axis=1)
    sq = (sm * sm).reshape(1, A)
    out_ref[0, 0] = jnp.concatenate([sq, jnp.zeros((7, A), jnp.float32)], axis=0)


_tc_kernel = pl.pallas_call(
    _tc_body,
    grid=(B_TC, 3),
    in_specs=[
        pl.BlockSpec((1, 1, A, D), lambda i, k: (B_SC + i, k, 0, 0)),
        pl.BlockSpec((1, 1, D), lambda i, k: (B_SC + i, 0, 0)),
    ],
    out_specs=pl.BlockSpec((1, 1, 8, A), lambda i, k: (i, k, 0, 0)),
    out_shape=jax.ShapeDtypeStruct((B_TC, 3, 8, A), jnp.float32),
)


def kernel(x, der_desc_wrt_pos):
    der4 = der_desc_wrt_pos.transpose(0, 3, 1, 2)
    der_flat = (
        der4.reshape(B * 3 * A // 8, 8, D // 128, 128)
        .transpose(0, 2, 1, 3)
        .reshape(-1)
    )
    sc_part = _sc_kernel(der_flat, x)
    tc_part = _tc_kernel(der4, x.reshape(B, 1, D))
    out_phys = jnp.concatenate(
        [sc_part.reshape(B_SC, 3, A), tc_part[:, :, 0, :]], axis=0
    )
    return out_phys.transpose(0, 2, 1).reshape(B, A * 3)
